# Initial kernel scaffold; baseline (speedup 1.0000x reference)
import jax
import jax.numpy as jnp
from jax.experimental import pallas as pl


def kernel(x, edge_index, batch, W1, b1, g1, be1, W2, b2, g2, be2, W3, b3, g3, be3):
    raise NotImplementedError("write your pallas kernel here")


# speedup vs baseline: 10.2218x; 10.2218x over previous
import functools

import jax
import jax.numpy as jnp
from jax import lax
from jax.experimental import pallas as pl
from jax.experimental.pallas import tpu as pltpu
from jax.experimental.pallas import tpu_sc as plsc

N = 10000
E = 320000
R = 10240
B = 1280
G = R // B
CHUNK = 128
NTILES = 16
ROWS_PER_TILE = R // NTILES

CPT = -(-E // (CHUNK * NTILES))
NCH = CPT * NTILES
EPAD = NCH * CHUNK

CPW0 = -(-E // (CHUNK * 2 * NTILES))
NCH0 = CPW0 * 2 * NTILES
EPAD0 = NCH0 * CHUNK


def _mesh():
    return plsc.VectorSubcoreMesh(core_axis_name="c", subcore_axis_name="s",
                                  num_cores=2, num_subcores=NTILES)


_SC_PARAMS = pltpu.CompilerParams(use_tc_tiling_on_sc=False)


@functools.cache
def _sc_degree_kernel():
    return functools.partial(
        pl.kernel,
        out_type=(jax.ShapeDtypeStruct((R, 16), jnp.float32),
                  jax.ShapeDtypeStruct((R, 16), jnp.float32)),
        mesh=_mesh(),
        scratch_types=[
            pltpu.VMEM((CHUNK,), jnp.int32),
            pltpu.VMEM((CHUNK, 16), jnp.float32),
            pltpu.VMEM_SHARED((R, 16), jnp.float32),
        ],
        compiler_params=_SC_PARAMS,
    )(_sc_degree_body)


def _sc_degree_body(dst_hbm, ones_hbm, zeros_hbm, dega, degb, dst_v, ones_v, acc):
    c = lax.axis_index("c")
    t = lax.axis_index("s")
    pltpu.sync_copy(zeros_hbm.at[:, :16], acc.at[pl.ds(t * ROWS_PER_TILE, ROWS_PER_TILE)])
    pltpu.sync_copy(ones_hbm, ones_v)
    plsc.subcore_barrier()
    w = c * NTILES + t

    @pl.loop(0, CPW0)
    def _body(j):
        ch = w * CPW0 + j
        pltpu.sync_copy(dst_hbm.at[ch], dst_v)
        pltpu.sync_copy(ones_v, acc.at[dst_v], add=True)

    plsc.subcore_barrier()
    sl = pl.ds(t * ROWS_PER_TILE, ROWS_PER_TILE)

    @pl.when(c == 0)
    def _():
        pltpu.sync_copy(acc.at[sl], dega.at[sl])

    @pl.when(c == 1)
    def _():
        pltpu.sync_copy(acc.at[sl], degb.at[sl])


@functools.cache
def _make_edge_agg(D):
    @functools.partial(
        pl.kernel,
        out_type=(jax.ShapeDtypeStruct((R, D), jnp.float32),
                  jax.ShapeDtypeStruct((R, D), jnp.float32)),
        mesh=_mesh(),
        scratch_types=[
            pltpu.VMEM((CHUNK,), jnp.int32),
            pltpu.VMEM((CHUNK,), jnp.int32),
            pltpu.VMEM((CHUNK, D), jnp.float32),
            pltpu.VMEM_SHARED((R, D), jnp.float32),
            pltpu.SemaphoreType.DMA,
        ],
        compiler_params=_SC_PARAMS,
    )
    def agg(ua, ub, src_hbm, dst_hbm, zeros_hbm, oa, ob,
            src_v, dst_v, rows_v, acc, sem):
        c = lax.axis_index("c")
        t = lax.axis_index("s")
        sl = pl.ds(t * ROWS_PER_TILE, ROWS_PER_TILE)
        pltpu.sync_copy(zeros_hbm.at[:, :D], acc.at[sl])
        plsc.subcore_barrier()

        @pl.loop(0, CPT)
        def _body(j):
            ch = t * CPT + j
            pltpu.sync_copy(src_hbm.at[ch], src_v)
            pltpu.sync_copy(dst_hbm.at[ch], dst_v)

            @pl.when(c == 0)
            def _():
                pltpu.async_copy(ua.at[src_v], rows_v, sem).wait()

            @pl.when(c == 1)
            def _():
                pltpu.async_copy(ub.at[src_v], rows_v, sem).wait()

            pltpu.sync_copy(rows_v, acc.at[dst_v], add=True)

        plsc.subcore_barrier()

        @pl.when(c == 0)
        def _():
            pltpu.sync_copy(acc.at[sl], oa.at[sl])

        @pl.when(c == 1)
        def _():
            pltpu.sync_copy(acc.at[sl], ob.at[sl])

    return agg


def _rowmask(i):
    rows = lax.broadcasted_iota(jnp.int32, (B, 1), 0) + i * B
    return (rows < N).astype(jnp.float32)


def _ka_body(x_ref, da_ref, db_ref, u1a_ref, u1b_ref, dinv_ref):
    deg = da_ref[:, :1] + db_ref[:, :1] + 1.0
    d = lax.rsqrt(deg)
    u = x_ref[...] * d
    u1a_ref[...] = u[:, :64]
    u1b_ref[...] = u[:, 64:]
    dinv_ref[...] = jnp.broadcast_to(d, (B, 128))


def _stats(m, i):
    msk = _rowmask(i)
    mm = m * msk
    s1 = jnp.sum(mm, axis=0).reshape(1, 1, -1)
    s2 = jnp.sum(mm * m, axis=0).reshape(1, 1, -1)
    return jnp.concatenate([s1, s2], axis=1)


def _kb_body(sa, sb, ua, ub, dv, w_ref, b_ref, m_ref, p_ref):
    s = jnp.concatenate([sa[...], sb[...]], axis=1)
    u = jnp.concatenate([ua[...], ub[...]], axis=1)
    a = (s + u) * dv[:, :1]
    m = jnp.dot(a, w_ref[...], preferred_element_type=jnp.float32) + b_ref[...]
    m_ref[...] = m
    p_ref[...] = _stats(m, pl.program_id(0))


def _bn_consts(p, g_ref, be_ref):
    ps = jnp.sum(p, axis=0)
    mu = ps[0] / N
    var = ps[1] / N - mu * mu
    scale = g_ref[0] * lax.rsqrt(var + 1e-5)
    shift = be_ref[0] - mu * scale
    return scale, shift


def _kc_body(m_ref, p_ref, dv, g_ref, be_ref, w_ref, ua_ref, ub_ref, *, D):
    scale, shift = _bn_consts(p_ref[...], g_ref, be_ref)
    h = jnp.maximum(m_ref[...] * scale + shift, 0.0)
    z = jnp.dot(h, w_ref[...], preferred_element_type=jnp.float32)
    u = z * dv[:, :1] * _rowmask(pl.program_id(0))
    ua_ref[...] = u[:, :D]
    ub_ref[...] = u[:, D:]


def _kd_body(sa, sb, ua, ub, dv, b_ref, m_ref, p_ref):
    s = jnp.concatenate([sa[...], sb[...]], axis=1)
    u = jnp.concatenate([ua[...], ub[...]], axis=1)
    m = (s + u) * dv[:, :1] + b_ref[...]
    m_ref[...] = m
    p_ref[...] = _stats(m, pl.program_id(0))


def _kf_body(sa, sb, ua, ub, dv, b_ref, bt_ref, p_ref, seg_ref, cnt_ref):
    s = jnp.concatenate([sa[...], sb[...]], axis=1)
    u = jnp.concatenate([ua[...], ub[...]], axis=1)
    m = (s + u) * dv[:, :1] + b_ref[...]
    p_ref[...] = _stats(m, pl.program_id(0))
    bt = bt_ref[:, :1]
    segs, cnts = [], []
    for g in range(16):
        mg = (bt == g).astype(jnp.float32)
        segs.append(jnp.sum(m * mg, axis=0).reshape(1, -1))
        cnts.append(jnp.broadcast_to(jnp.sum(mg, axis=0).reshape(1, 1), (1, 128)))
    seg_ref[...] = jnp.concatenate(segs, axis=0).reshape(1, 16, 128)
    cnt_ref[...] = jnp.concatenate(cnts, axis=0).reshape(1, 16, 128)


def _kg_body(p_ref, seg_ref, cnt_ref, g_ref, be_ref, out_ref):
    scale, shift = _bn_consts(p_ref[...], g_ref, be_ref)
    sums = jnp.sum(seg_ref[...], axis=0)
    cnt = jnp.sum(cnt_ref[...], axis=0)
    pool = sums / jnp.maximum(cnt, 1.0)
    out_ref[...] = jnp.where(cnt > 0, pool * scale + shift, 0.0)


def _rb(d):
    return pl.BlockSpec((B, d), lambda i: (i, 0))


def _full2(shape):
    return pl.BlockSpec(shape, lambda i: (0, 0))


def _pspec(d):
    return pl.BlockSpec((1, 2, d), lambda i: (i, 0, 0))


def _f32(*shape):
    return jax.ShapeDtypeStruct(shape, jnp.float32)


def kernel(x, edge_index, batch, W1, b1, g1, be1, W2, b2, g2, be2,
           W3, b3, g3, be3):
    f32 = jnp.float32
    src = edge_index[0]
    dst = edge_index[1]

    x_pad = jnp.pad(x, ((0, R - N), (0, 0)))
    batch_pad = jnp.broadcast_to(
        jnp.pad(batch, (0, R - N), constant_values=16)[:, None], (R, 16))

    fill = (N + (jnp.arange(EPAD - E, dtype=jnp.int32) % 16))
    src_p = jnp.concatenate([src, fill]).reshape(NCH, CHUNK)
    dst_p = jnp.concatenate([dst, fill]).reshape(NCH, CHUNK)
    fill0 = (N + (jnp.arange(EPAD0 - E, dtype=jnp.int32) % 16))
    dst0 = jnp.concatenate([dst, fill0]).reshape(NCH0, CHUNK)

    ones16 = jnp.ones((CHUNK, 16), f32)
    zrows = jnp.zeros((ROWS_PER_TILE, 128), f32)

    dega, degb = _sc_degree_kernel()(dst0, ones16, zrows)

    u1a, u1b, dinv = pl.pallas_call(
        _ka_body,
        grid=(G,),
        in_specs=[_rb(128), _rb(16), _rb(16)],
        out_specs=[_rb(64), _rb(64), _rb(128)],
        out_shape=[_f32(R, 64), _f32(R, 64), _f32(R, 128)],
    )(x_pad, dega, degb)

    s1a, s1b = _make_edge_agg(64)(u1a, u1b, src_p, dst_p, zrows)

    m1, p1 = pl.pallas_call(
        _kb_body,
        grid=(G,),
        in_specs=[_rb(64), _rb(64), _rb(64), _rb(64), _rb(128),
                  _full2((128, 256)), _full2((1, 256))],
        out_specs=[_rb(256), _pspec(256)],
        out_shape=[_f32(R, 256), _f32(G, 2, 256)],
    )(s1a, s1b, u1a, u1b, dinv, W1, b1.reshape(1, 256))

    u2a, u2b = pl.pallas_call(
        functools.partial(_kc_body, D=128),
        grid=(G,),
        in_specs=[_rb(256), pl.BlockSpec((G, 2, 256), lambda i: (0, 0, 0)),
                  _rb(128), _full2((1, 256)), _full2((1, 256)),
                  _full2((256, 256))],
        out_specs=[_rb(128), _rb(128)],
        out_shape=[_f32(R, 128), _f32(R, 128)],
    )(m1, p1, dinv, g1.reshape(1, 256), be1.reshape(1, 256), W2)

    s2a, s2b = _make_edge_agg(128)(u2a, u2b, src_p, dst_p, zrows)

    m2, p2 = pl.pallas_call(
        _kd_body,
        grid=(G,),
        in_specs=[_rb(128), _rb(128), _rb(128), _rb(128), _rb(128),
                  _full2((1, 256))],
        out_specs=[_rb(256), _pspec(256)],
        out_shape=[_f32(R, 256), _f32(G, 2, 256)],
    )(s2a, s2b, u2a, u2b, dinv, b2.reshape(1, 256))

    u3a, u3b = pl.pallas_call(
        functools.partial(_kc_body, D=64),
        grid=(G,),
        in_specs=[_rb(256), pl.BlockSpec((G, 2, 256), lambda i: (0, 0, 0)),
                  _rb(128), _full2((1, 256)), _full2((1, 256)),
                  _full2((256, 128))],
        out_specs=[_rb(64), _rb(64)],
        out_shape=[_f32(R, 64), _f32(R, 64)],
    )(m2, p2, dinv, g2.reshape(1, 256), be2.reshape(1, 256), W3)

    s3a, s3b = _make_edge_agg(64)(u3a, u3b, src_p, dst_p, zrows)

    p3, segp, cntp = pl.pallas_call(
        _kf_body,
        grid=(G,),
        in_specs=[_rb(64), _rb(64), _rb(64), _rb(64), _rb(128),
                  _full2((1, 128)),
                  pl.BlockSpec((B, 16), lambda i: (i, 0))],
        out_specs=[_pspec(128),
                   pl.BlockSpec((1, 16, 128), lambda i: (i, 0, 0)),
                   pl.BlockSpec((1, 16, 128), lambda i: (i, 0, 0))],
        out_shape=[_f32(G, 2, 128), _f32(G, 16, 128), _f32(G, 16, 128)],
    )(s3a, s3b, u3a, u3b, dinv, b3.reshape(1, 128), batch_pad)

    out = pl.pallas_call(
        _kg_body,
        grid=(1,),
        in_specs=[pl.BlockSpec((G, 2, 128), lambda i: (0, 0, 0)),
                  pl.BlockSpec((G, 16, 128), lambda i: (0, 0, 0)),
                  pl.BlockSpec((G, 16, 128), lambda i: (0, 0, 0)),
                  _full2((1, 128)), _full2((1, 128))],
        out_specs=pl.BlockSpec((16, 128), lambda i: (0, 0)),
        out_shape=_f32(16, 128),
    )(p3, segp, cntp, g3.reshape(1, 128), be3.reshape(1, 128))

    return out


# speedup vs baseline: 17.4107x; 1.7033x over previous
import functools

import jax
import jax.numpy as jnp
from jax import lax
from jax.experimental import pallas as pl
from jax.experimental.pallas import tpu as pltpu
from jax.experimental.pallas import tpu_sc as plsc

N = 10000
E = 320000
R = 10240
B = 1280
G = R // B
CHUNK = 128
NTILES = 16
ROWS_PER_TILE = R // NTILES

CPT = 160
NCH = CPT * NTILES
EPAD = NCH * CHUNK
CPW0 = NCH // 32


def _mesh():
    return plsc.VectorSubcoreMesh(core_axis_name="c", subcore_axis_name="s",
                                  num_cores=2, num_subcores=NTILES)


_SC_PARAMS = pltpu.CompilerParams(use_tc_tiling_on_sc=False)


_NB0 = 4


@functools.cache
def _sc_degree_kernel():
    return functools.partial(
        pl.kernel,
        out_type=jax.ShapeDtypeStruct((2, R, 8), jnp.float32),
        mesh=_mesh(),
        scratch_types=[
            pltpu.VMEM((CPW0, CHUNK), jnp.int32),
            pltpu.VMEM((CHUNK, 8), jnp.float32),
            pltpu.VMEM_SHARED((R, 8), jnp.float32),
        ] + [pltpu.SemaphoreType.DMA] * _NB0,
        compiler_params=_SC_PARAMS,
    )(_sc_degree_body)


def _sc_degree_body(dst_hbm, ones_hbm, zeros_hbm, deg_out,
                    dst_all, ones_v, acc, *sems):
    c = lax.axis_index("c")
    t = lax.axis_index("s")
    sl = pl.ds(t * ROWS_PER_TILE, ROWS_PER_TILE)
    w = c * NTILES + t
    pltpu.sync_copy(zeros_hbm, acc.at[sl])
    pltpu.sync_copy(ones_hbm, ones_v)
    pltpu.sync_copy(dst_hbm.at[pl.ds(w * CPW0, CPW0)], dst_all)
    plsc.subcore_barrier()

    @pl.loop(0, CPW0 // _NB0)
    def _body(g):
        ds = []
        for b in range(_NB0):
            ds.append(pltpu.async_copy(
                ones_v, acc.at[dst_all.at[g * _NB0 + b]], sems[b], add=True))
        for d in ds:
            d.wait()

    plsc.subcore_barrier()
    pltpu.sync_copy(acc.at[sl], deg_out.at[c, sl])


_NB = 5
_NG = CPT // _NB


@functools.cache
def _edge_agg_kernel():
    @functools.partial(
        pl.kernel,
        out_type=jax.ShapeDtypeStruct((2, R, 64), jnp.float32),
        mesh=_mesh(),
        scratch_types=[
            pltpu.VMEM((CPT, CHUNK), jnp.int32),
            pltpu.VMEM((CPT, CHUNK), jnp.int32),
            pltpu.VMEM((_NB, CHUNK, 64), jnp.float32),
            pltpu.VMEM_SHARED((R, 64), jnp.float32),
        ] + [pltpu.SemaphoreType.DMA] * (2 * _NB),
        compiler_params=_SC_PARAMS,
    )
    def agg(ua, ub, src_hbm, dst_hbm, zeros_hbm, out,
            src_all, dst_all, rows_v, acc, *sems):
        gsem, ssem = sems[:_NB], sems[_NB:]
        c = lax.axis_index("c")
        t = lax.axis_index("s")
        sl = pl.ds(t * ROWS_PER_TILE, ROWS_PER_TILE)
        pltpu.sync_copy(zeros_hbm, acc.at[sl])
        pltpu.sync_copy(src_hbm.at[pl.ds(t * CPT, CPT)], src_all)
        pltpu.sync_copy(dst_hbm.at[pl.ds(t * CPT, CPT)], dst_all)
        plsc.subcore_barrier()

        def run(uref):
            @pl.loop(0, _NG)
            def _body(g):
                base = g * _NB
                gds = [pltpu.async_copy(uref.at[src_all.at[base + b]],
                                        rows_v.at[b], gsem[b])
                       for b in range(_NB)]
                sds = []
                for b in range(_NB):
                    gds[b].wait()
                    sds.append(pltpu.async_copy(
                        rows_v.at[b], acc.at[dst_all.at[base + b]],
                        ssem[b], add=True))
                for d in sds:
                    d.wait()

        @pl.when(c == 0)
        def _():
            run(ua)

        @pl.when(c == 1)
        def _():
            run(ub)

        plsc.subcore_barrier()
        pltpu.sync_copy(acc.at[sl], out.at[c, sl])

    return agg


def _rowmask(i):
    rows = lax.broadcasted_iota(jnp.int32, (B, 1), 0) + i * B
    return (rows < N).astype(jnp.float32)


def _ka_body(x_ref, deg_ref, u1a_ref, u1b_ref, dinv_ref):
    deg = deg_ref[0][:, :1] + deg_ref[1][:, :1] + 1.0
    d = lax.rsqrt(deg)
    u = x_ref[...] * d
    u1a_ref[...] = u[:, :64]
    u1b_ref[...] = u[:, 64:]
    dinv_ref[...] = jnp.broadcast_to(d, (B, 128))


def _stats(m, i):
    msk = _rowmask(i)
    mm = m * msk
    s1 = jnp.sum(mm, axis=0).reshape(1, 1, -1)
    s2 = jnp.sum(mm * m, axis=0).reshape(1, 1, -1)
    return jnp.concatenate([s1, s2], axis=1)


def _kb_body(sa, sb, ua, ub, dv, w_ref, b_ref, m_ref, p_ref):
    s = jnp.concatenate([sa[...], sb[...]], axis=1)
    u = jnp.concatenate([ua[...], ub[...]], axis=1)
    a = (s + u) * dv[:, :1]
    m = jnp.dot(a, w_ref[...], preferred_element_type=jnp.float32) + b_ref[...]
    m_ref[...] = m
    p_ref[...] = _stats(m, pl.program_id(0))


def _bn_consts(p, g_ref, be_ref):
    ps = jnp.sum(p, axis=0)
    mu = ps[0] / N
    var = ps[1] / N - mu * mu
    scale = g_ref[0] * lax.rsqrt(var + 1e-5)
    shift = be_ref[0] - mu * scale
    return scale, shift


def _kc_body(m_ref, p_ref, dv, g_ref, be_ref, w_ref, *urefs):
    scale, shift = _bn_consts(p_ref[...], g_ref, be_ref)
    h = jnp.maximum(m_ref[...] * scale + shift, 0.0)
    z = jnp.dot(h, w_ref[...], preferred_element_type=jnp.float32)
    u = z * dv[:, :1] * _rowmask(pl.program_id(0))
    for k, ur in enumerate(urefs):
        ur[...] = u[:, 64 * k:64 * (k + 1)]


def _kd_body(sq0, sq1, sq2, sq3, uq0, uq1, uq2, uq3, dv, b_ref, m_ref, p_ref):
    s = jnp.concatenate([sq0[...], sq1[...], sq2[...], sq3[...]], axis=1)
    u = jnp.concatenate([uq0[...], uq1[...], uq2[...], uq3[...]], axis=1)
    m = (s + u) * dv[:, :1] + b_ref[...]
    m_ref[...] = m
    p_ref[...] = _stats(m, pl.program_id(0))


def _kf_body(sa, sb, ua, ub, dv, b_ref, bt_ref, p_ref, seg_ref, cnt_ref):
    s = jnp.concatenate([sa[...], sb[...]], axis=1)
    u = jnp.concatenate([ua[...], ub[...]], axis=1)
    m = (s + u) * dv[:, :1] + b_ref[...]
    p_ref[...] = _stats(m, pl.program_id(0))
    bt = bt_ref[:, :1]
    segs, cnts = [], []
    for g in range(16):
        mg = (bt == g).astype(jnp.float32)
        segs.append(jnp.sum(m * mg, axis=0).reshape(1, -1))
        cnts.append(jnp.broadcast_to(jnp.sum(mg, axis=0).reshape(1, 1), (1, 128)))
    seg_ref[...] = jnp.concatenate(segs, axis=0).reshape(1, 16, 128)
    cnt_ref[...] = jnp.concatenate(cnts, axis=0).reshape(1, 16, 128)


def _kg_body(p_ref, seg_ref, cnt_ref, g_ref, be_ref, out_ref):
    scale, shift = _bn_consts(p_ref[...], g_ref, be_ref)
    sums = jnp.sum(seg_ref[...], axis=0)
    cnt = jnp.sum(cnt_ref[...], axis=0)
    pool = sums / jnp.maximum(cnt, 1.0)
    out_ref[...] = jnp.where(cnt > 0, pool * scale + shift, 0.0)


def _rb(d):
    return pl.BlockSpec((B, d), lambda i: (i, 0))


def _full2(shape):
    return pl.BlockSpec(shape, lambda i: (0, 0))


def _pspec(d):
    return pl.BlockSpec((1, 2, d), lambda i: (i, 0, 0))


def _f32(*shape):
    return jax.ShapeDtypeStruct(shape, jnp.float32)


def kernel(x, edge_index, batch, W1, b1, g1, be1, W2, b2, g2, be2,
           W3, b3, g3, be3):
    f32 = jnp.float32
    src = edge_index[0]
    dst = edge_index[1]

    x_pad = jnp.pad(x, ((0, R - N), (0, 0)))
    batch_pad = jnp.broadcast_to(
        jnp.pad(batch, (0, R - N), constant_values=16)[:, None], (R, 16))

    fill = (N + (jnp.arange(EPAD - E, dtype=jnp.int32) % 16))
    src_p = jnp.concatenate([src, fill]).reshape(NCH, CHUNK)
    dst_p = jnp.concatenate([dst, fill]).reshape(NCH, CHUNK)

    ones8 = jnp.ones((CHUNK, 8), f32)
    z8 = jnp.zeros((ROWS_PER_TILE, 8), f32)
    z64 = jnp.zeros((ROWS_PER_TILE, 64), f32)

    deg2 = _sc_degree_kernel()(dst_p, ones8, z8)

    u1a, u1b, dinv = pl.pallas_call(
        _ka_body,
        grid=(G,),
        in_specs=[_rb(128), pl.BlockSpec((2, B, 8), lambda i: (0, i, 0))],
        out_specs=[_rb(64), _rb(64), _rb(128)],
        out_shape=[_f32(R, 64), _f32(R, 64), _f32(R, 128)],
    )(x_pad, deg2)

    s1 = _edge_agg_kernel()(u1a, u1b, src_p, dst_p, z64)
    s1a, s1b = s1[0], s1[1]

    m1, p1 = pl.pallas_call(
        _kb_body,
        grid=(G,),
        in_specs=[_rb(64), _rb(64), _rb(64), _rb(64), _rb(128),
                  _full2((128, 256)), _full2((1, 256))],
        out_specs=[_rb(256), _pspec(256)],
        out_shape=[_f32(R, 256), _f32(G, 2, 256)],
    )(s1a, s1b, u1a, u1b, dinv, W1, b1.reshape(1, 256))

    u2q = pl.pallas_call(
        _kc_body,
        grid=(G,),
        in_specs=[_rb(256), pl.BlockSpec((G, 2, 256), lambda i: (0, 0, 0)),
                  _rb(128), _full2((1, 256)), _full2((1, 256)),
                  _full2((256, 256))],
        out_specs=[_rb(64)] * 4,
        out_shape=[_f32(R, 64)] * 4,
    )(m1, p1, dinv, g1.reshape(1, 256), be1.reshape(1, 256), W2)

    s02 = _edge_agg_kernel()(u2q[0], u2q[2], src_p, dst_p, z64)
    s13 = _edge_agg_kernel()(u2q[1], u2q[3], src_p, dst_p, z64)

    m2, p2 = pl.pallas_call(
        _kd_body,
        grid=(G,),
        in_specs=[_rb(64)] * 8 + [_rb(128), _full2((1, 256))],
        out_specs=[_rb(256), _pspec(256)],
        out_shape=[_f32(R, 256), _f32(G, 2, 256)],
    )(s02[0], s13[0], s02[1], s13[1], u2q[0], u2q[1], u2q[2], u2q[3],
      dinv, b2.reshape(1, 256))

    u3a, u3b = pl.pallas_call(
        _kc_body,
        grid=(G,),
        in_specs=[_rb(256), pl.BlockSpec((G, 2, 256), lambda i: (0, 0, 0)),
                  _rb(128), _full2((1, 256)), _full2((1, 256)),
                  _full2((256, 128))],
        out_specs=[_rb(64)] * 2,
        out_shape=[_f32(R, 64)] * 2,
    )(m2, p2, dinv, g2.reshape(1, 256), be2.reshape(1, 256), W3)

    s3 = _edge_agg_kernel()(u3a, u3b, src_p, dst_p, z64)
    s3a, s3b = s3[0], s3[1]

    p3, segp, cntp = pl.pallas_call(
        _kf_body,
        grid=(G,),
        in_specs=[_rb(64), _rb(64), _rb(64), _rb(64), _rb(128),
                  _full2((1, 128)),
                  pl.BlockSpec((B, 16), lambda i: (i, 0))],
        out_specs=[_pspec(128),
                   pl.BlockSpec((1, 16, 128), lambda i: (i, 0, 0)),
                   pl.BlockSpec((1, 16, 128), lambda i: (i, 0, 0))],
        out_shape=[_f32(G, 2, 128), _f32(G, 16, 128), _f32(G, 16, 128)],
    )(s3a, s3b, u3a, u3b, dinv, b3.reshape(1, 128), batch_pad)

    out = pl.pallas_call(
        _kg_body,
        grid=(1,),
        in_specs=[pl.BlockSpec((G, 2, 128), lambda i: (0, 0, 0)),
                  pl.BlockSpec((G, 16, 128), lambda i: (0, 0, 0)),
                  pl.BlockSpec((G, 16, 128), lambda i: (0, 0, 0)),
                  _full2((1, 128)), _full2((1, 128))],
        out_specs=pl.BlockSpec((16, 128), lambda i: (0, 0)),
        out_shape=_f32(16, 128),
    )(p3, segp, cntp, g3.reshape(1, 128), be3.reshape(1, 128))

    return out


# speedup vs baseline: 20.6071x; 1.1836x over previous
import functools

import jax
import jax.numpy as jnp
from jax import lax
from jax.experimental import pallas as pl
from jax.experimental.pallas import tpu as pltpu
from jax.experimental.pallas import tpu_sc as plsc

N = 10000
E = 320000
R = 10240
B = 1280
G = R // B
CHUNK = 128
NTILES = 16
ROWS_PER_TILE = R // NTILES

CPT = 160
NCH = CPT * NTILES
EPAD = NCH * CHUNK
CPW0 = NCH // 32


def _mesh():
    return plsc.VectorSubcoreMesh(core_axis_name="c", subcore_axis_name="s",
                                  num_cores=2, num_subcores=NTILES)


_SC_PARAMS = pltpu.CompilerParams(use_tc_tiling_on_sc=False)


_NB0 = 4


@functools.cache
def _sc_degree_kernel():
    return functools.partial(
        pl.kernel,
        out_type=jax.ShapeDtypeStruct((2, R, 8), jnp.float32),
        mesh=_mesh(),
        scratch_types=[
            pltpu.VMEM((CPW0, CHUNK), jnp.int32),
            pltpu.VMEM((CHUNK, 8), jnp.float32),
            pltpu.VMEM_SHARED((R, 8), jnp.float32),
        ] + [pltpu.SemaphoreType.DMA] * _NB0,
        compiler_params=_SC_PARAMS,
    )(_sc_degree_body)


def _sc_degree_body(dst_hbm, ones_hbm, zeros_hbm, deg_out,
                    dst_all, ones_v, acc, *sems):
    c = lax.axis_index("c")
    t = lax.axis_index("s")
    sl = pl.ds(t * ROWS_PER_TILE, ROWS_PER_TILE)
    w = c * NTILES + t
    pltpu.sync_copy(zeros_hbm, acc.at[sl])
    pltpu.sync_copy(ones_hbm, ones_v)
    pltpu.sync_copy(dst_hbm.at[pl.ds(w * CPW0, CPW0)], dst_all)
    plsc.subcore_barrier()

    @pl.loop(0, CPW0 // _NB0)
    def _body(g):
        ds = []
        for b in range(_NB0):
            ds.append(pltpu.async_copy(
                ones_v, acc.at[dst_all.at[g * _NB0 + b]], sems[b], add=True))
        for d in ds:
            d.wait()

    plsc.subcore_barrier()
    pltpu.sync_copy(acc.at[sl], deg_out.at[c, sl])


_NB = 5
_NG = CPT // _NB


@functools.cache
def _edge_agg_kernel():
    @functools.partial(
        pl.kernel,
        out_type=jax.ShapeDtypeStruct((2, R, 64), jnp.float32),
        mesh=_mesh(),
        scratch_types=[
            pltpu.VMEM((CPT, CHUNK), jnp.int32),
            pltpu.VMEM((CPT, CHUNK), jnp.int32),
            pltpu.VMEM((_NB, CHUNK, 64), jnp.float32),
            pltpu.VMEM_SHARED((R, 64), jnp.float32),
        ] + [pltpu.SemaphoreType.DMA] * (2 * _NB),
        compiler_params=_SC_PARAMS,
    )
    def agg(ua, ub, src_hbm, dst_hbm, zeros_hbm, out,
            src_all, dst_all, rows_v, acc, *sems):
        gsem, ssem = sems[:_NB], sems[_NB:]
        c = lax.axis_index("c")
        t = lax.axis_index("s")
        sl = pl.ds(t * ROWS_PER_TILE, ROWS_PER_TILE)
        pltpu.sync_copy(zeros_hbm, acc.at[sl])
        pltpu.sync_copy(src_hbm.at[pl.ds(t * CPT, CPT)], src_all)
        pltpu.sync_copy(dst_hbm.at[pl.ds(t * CPT, CPT)], dst_all)
        plsc.subcore_barrier()

        def run(uref):
            for b in range(_NB):
                pltpu.async_copy(uref.at[src_all.at[b]], rows_v.at[b], gsem[b])

            @pl.loop(0, _NG - 1)
            def _body(g):
                base = g * _NB
                for b in range(_NB):
                    c = base + b
                    pltpu.make_async_copy(uref.at[src_all.at[c]],
                                          rows_v.at[b], gsem[b]).wait()
                    pltpu.async_copy(rows_v.at[b], acc.at[dst_all.at[c]],
                                     ssem[b], add=True)
                    pltpu.make_async_copy(rows_v.at[b],
                                          acc.at[dst_all.at[c]],
                                          ssem[b]).wait()
                    pltpu.async_copy(uref.at[src_all.at[c + _NB]],
                                     rows_v.at[b], gsem[b])

            base = (_NG - 1) * _NB
            sds = []
            for b in range(_NB):
                pltpu.make_async_copy(uref.at[src_all.at[base + b]],
                                      rows_v.at[b], gsem[b]).wait()
                sds.append(pltpu.async_copy(
                    rows_v.at[b], acc.at[dst_all.at[base + b]],
                    ssem[b], add=True))
            for d in sds:
                d.wait()

        @pl.when(c == 0)
        def _():
            run(ua)

        @pl.when(c == 1)
        def _():
            run(ub)

        plsc.subcore_barrier()
        pltpu.sync_copy(acc.at[sl], out.at[c, sl])

    return agg


def _rowmask(i):
    rows = lax.broadcasted_iota(jnp.int32, (B, 1), 0) + i * B
    return (rows < N).astype(jnp.float32)


def _ka_body(x_ref, deg_ref, u1a_ref, u1b_ref, dinv_ref):
    deg = deg_ref[0][:, :1] + deg_ref[1][:, :1] + 1.0
    d = lax.rsqrt(deg)
    u = x_ref[...] * d
    u1a_ref[...] = u[:, :64]
    u1b_ref[...] = u[:, 64:]
    dinv_ref[...] = jnp.broadcast_to(d, (B, 128))


def _stats(m, i):
    msk = _rowmask(i)
    mm = m * msk
    s1 = jnp.sum(mm, axis=0).reshape(1, 1, -1)
    s2 = jnp.sum(mm * m, axis=0).reshape(1, 1, -1)
    return jnp.concatenate([s1, s2], axis=1)


def _kb_body(sa, sb, ua, ub, dv, w_ref, b_ref, m_ref, p_ref):
    s = jnp.concatenate([sa[...], sb[...]], axis=1)
    u = jnp.concatenate([ua[...], ub[...]], axis=1)
    a = (s + u) * dv[:, :1]
    m = jnp.dot(a, w_ref[...], preferred_element_type=jnp.float32) + b_ref[...]
    m_ref[...] = m
    p_ref[...] = _stats(m, pl.program_id(0))


def _bn_consts(p, g_ref, be_ref):
    ps = jnp.sum(p, axis=0)
    mu = ps[0] / N
    var = ps[1] / N - mu * mu
    scale = g_ref[0] * lax.rsqrt(var + 1e-5)
    shift = be_ref[0] - mu * scale
    return scale, shift


def _kc_body(m_ref, p_ref, dv, g_ref, be_ref, w_ref, *urefs):
    scale, shift = _bn_consts(p_ref[...], g_ref, be_ref)
    h = jnp.maximum(m_ref[...] * scale + shift, 0.0)
    z = jnp.dot(h, w_ref[...], preferred_element_type=jnp.float32)
    u = z * dv[:, :1] * _rowmask(pl.program_id(0))
    for k, ur in enumerate(urefs):
        ur[...] = u[:, 64 * k:64 * (k + 1)]


def _kd_body(sq0, sq1, sq2, sq3, uq0, uq1, uq2, uq3, dv, b_ref, m_ref, p_ref):
    s = jnp.concatenate([sq0[...], sq1[...], sq2[...], sq3[...]], axis=1)
    u = jnp.concatenate([uq0[...], uq1[...], uq2[...], uq3[...]], axis=1)
    m = (s + u) * dv[:, :1] + b_ref[...]
    m_ref[...] = m
    p_ref[...] = _stats(m, pl.program_id(0))


def _kf_body(sa, sb, ua, ub, dv, b_ref, bt_ref, p_ref, seg_ref, cnt_ref):
    s = jnp.concatenate([sa[...], sb[...]], axis=1)
    u = jnp.concatenate([ua[...], ub[...]], axis=1)
    m = (s + u) * dv[:, :1] + b_ref[...]
    p_ref[...] = _stats(m, pl.program_id(0))
    bt = bt_ref[:, :1]
    segs, cnts = [], []
    for g in range(16):
        mg = (bt == g).astype(jnp.float32)
        segs.append(jnp.sum(m * mg, axis=0).reshape(1, -1))
        cnts.append(jnp.broadcast_to(jnp.sum(mg, axis=0).reshape(1, 1), (1, 128)))
    seg_ref[...] = jnp.concatenate(segs, axis=0).reshape(1, 16, 128)
    cnt_ref[...] = jnp.concatenate(cnts, axis=0).reshape(1, 16, 128)


def _kg_body(p_ref, seg_ref, cnt_ref, g_ref, be_ref, out_ref):
    scale, shift = _bn_consts(p_ref[...], g_ref, be_ref)
    sums = jnp.sum(seg_ref[...], axis=0)
    cnt = jnp.sum(cnt_ref[...], axis=0)
    pool = sums / jnp.maximum(cnt, 1.0)
    out_ref[...] = jnp.where(cnt > 0, pool * scale + shift, 0.0)


def _rb(d):
    return pl.BlockSpec((B, d), lambda i: (i, 0))


def _full2(shape):
    return pl.BlockSpec(shape, lambda i: (0, 0))


def _pspec(d):
    return pl.BlockSpec((1, 2, d), lambda i: (i, 0, 0))


def _f32(*shape):
    return jax.ShapeDtypeStruct(shape, jnp.float32)


def kernel(x, edge_index, batch, W1, b1, g1, be1, W2, b2, g2, be2,
           W3, b3, g3, be3):
    f32 = jnp.float32
    src = edge_index[0]
    dst = edge_index[1]

    x_pad = jnp.pad(x, ((0, R - N), (0, 0)))
    batch_pad = jnp.broadcast_to(
        jnp.pad(batch, (0, R - N), constant_values=16)[:, None], (R, 16))

    fill = (N + (jnp.arange(EPAD - E, dtype=jnp.int32) % 16))
    src_p = jnp.concatenate([src, fill]).reshape(NCH, CHUNK)
    dst_p = jnp.concatenate([dst, fill]).reshape(NCH, CHUNK)

    ones8 = jnp.ones((CHUNK, 8), f32)
    z8 = jnp.zeros((ROWS_PER_TILE, 8), f32)
    z64 = jnp.zeros((ROWS_PER_TILE, 64), f32)

    deg2 = _sc_degree_kernel()(dst_p, ones8, z8)

    u1a, u1b, dinv = pl.pallas_call(
        _ka_body,
        grid=(G,),
        in_specs=[_rb(128), pl.BlockSpec((2, B, 8), lambda i: (0, i, 0))],
        out_specs=[_rb(64), _rb(64), _rb(128)],
        out_shape=[_f32(R, 64), _f32(R, 64), _f32(R, 128)],
    )(x_pad, deg2)

    s1 = _edge_agg_kernel()(u1a, u1b, src_p, dst_p, z64)
    s1a, s1b = s1[0], s1[1]

    m1, p1 = pl.pallas_call(
        _kb_body,
        grid=(G,),
        in_specs=[_rb(64), _rb(64), _rb(64), _rb(64), _rb(128),
                  _full2((128, 256)), _full2((1, 256))],
        out_specs=[_rb(256), _pspec(256)],
        out_shape=[_f32(R, 256), _f32(G, 2, 256)],
    )(s1a, s1b, u1a, u1b, dinv, W1, b1.reshape(1, 256))

    u2q = pl.pallas_call(
        _kc_body,
        grid=(G,),
        in_specs=[_rb(256), pl.BlockSpec((G, 2, 256), lambda i: (0, 0, 0)),
                  _rb(128), _full2((1, 256)), _full2((1, 256)),
                  _full2((256, 256))],
        out_specs=[_rb(64)] * 4,
        out_shape=[_f32(R, 64)] * 4,
    )(m1, p1, dinv, g1.reshape(1, 256), be1.reshape(1, 256), W2)

    s02 = _edge_agg_kernel()(u2q[0], u2q[2], src_p, dst_p, z64)
    s13 = _edge_agg_kernel()(u2q[1], u2q[3], src_p, dst_p, z64)

    m2, p2 = pl.pallas_call(
        _kd_body,
        grid=(G,),
        in_specs=[_rb(64)] * 8 + [_rb(128), _full2((1, 256))],
        out_specs=[_rb(256), _pspec(256)],
        out_shape=[_f32(R, 256), _f32(G, 2, 256)],
    )(s02[0], s13[0], s02[1], s13[1], u2q[0], u2q[1], u2q[2], u2q[3],
      dinv, b2.reshape(1, 256))

    u3a, u3b = pl.pallas_call(
        _kc_body,
        grid=(G,),
        in_specs=[_rb(256), pl.BlockSpec((G, 2, 256), lambda i: (0, 0, 0)),
                  _rb(128), _full2((1, 256)), _full2((1, 256)),
                  _full2((256, 128))],
        out_specs=[_rb(64)] * 2,
        out_shape=[_f32(R, 64)] * 2,
    )(m2, p2, dinv, g2.reshape(1, 256), be2.reshape(1, 256), W3)

    s3 = _edge_agg_kernel()(u3a, u3b, src_p, dst_p, z64)
    s3a, s3b = s3[0], s3[1]

    p3, segp, cntp = pl.pallas_call(
        _kf_body,
        grid=(G,),
        in_specs=[_rb(64), _rb(64), _rb(64), _rb(64), _rb(128),
                  _full2((1, 128)),
                  pl.BlockSpec((B, 16), lambda i: (i, 0))],
        out_specs=[_pspec(128),
                   pl.BlockSpec((1, 16, 128), lambda i: (i, 0, 0)),
                   pl.BlockSpec((1, 16, 128), lambda i: (i, 0, 0))],
        out_shape=[_f32(G, 2, 128), _f32(G, 16, 128), _f32(G, 16, 128)],
    )(s3a, s3b, u3a, u3b, dinv, b3.reshape(1, 128), batch_pad)

    out = pl.pallas_call(
        _kg_body,
        grid=(1,),
        in_specs=[pl.BlockSpec((G, 2, 128), lambda i: (0, 0, 0)),
                  pl.BlockSpec((G, 16, 128), lambda i: (0, 0, 0)),
                  pl.BlockSpec((G, 16, 128), lambda i: (0, 0, 0)),
                  _full2((1, 128)), _full2((1, 128))],
        out_specs=pl.BlockSpec((16, 128), lambda i: (0, 0)),
        out_shape=_f32(16, 128),
    )(p3, segp, cntp, g3.reshape(1, 128), be3.reshape(1, 128))

    return out


# speedup vs baseline: 21.2868x; 1.0330x over previous
import functools

import jax
import jax.numpy as jnp
from jax import lax
from jax.experimental import pallas as pl
from jax.experimental.pallas import tpu as pltpu
from jax.experimental.pallas import tpu_sc as plsc

N = 10000
E = 320000
R = 10240
B = 1280
G = R // B
CHUNK = 128
NTILES = 16
ROWS_PER_TILE = R // NTILES

CPT = 160
NCH = CPT * NTILES
EPAD = NCH * CHUNK
CPW0 = NCH // 32


def _mesh():
    return plsc.VectorSubcoreMesh(core_axis_name="c", subcore_axis_name="s",
                                  num_cores=2, num_subcores=NTILES)


_SC_PARAMS = pltpu.CompilerParams(use_tc_tiling_on_sc=False)


_NB0 = 4


@functools.cache
def _sc_degree_kernel():
    return functools.partial(
        pl.kernel,
        out_type=jax.ShapeDtypeStruct((2, R, 8), jnp.float32),
        mesh=_mesh(),
        scratch_types=[
            pltpu.VMEM((CPW0, CHUNK), jnp.int32),
            pltpu.VMEM((CHUNK, 8), jnp.float32),
            pltpu.VMEM_SHARED((R, 8), jnp.float32),
        ] + [pltpu.SemaphoreType.DMA] * _NB0,
        compiler_params=_SC_PARAMS,
    )(_sc_degree_body)


def _sc_degree_body(dst_hbm, ones_hbm, zeros_hbm, deg_out,
                    dst_all, ones_v, acc, *sems):
    c = lax.axis_index("c")
    t = lax.axis_index("s")
    sl = pl.ds(t * ROWS_PER_TILE, ROWS_PER_TILE)
    w = c * NTILES + t
    pltpu.sync_copy(zeros_hbm, acc.at[sl])
    pltpu.sync_copy(ones_hbm, ones_v)
    pltpu.sync_copy(dst_hbm.at[pl.ds(w * CPW0, CPW0)], dst_all)
    plsc.subcore_barrier()

    @pl.loop(0, CPW0 // _NB0)
    def _body(g):
        ds = []
        for b in range(_NB0):
            ds.append(pltpu.async_copy(
                ones_v, acc.at[dst_all.at[g * _NB0 + b]], sems[b], add=True))
        for d in ds:
            d.wait()

    plsc.subcore_barrier()
    pltpu.sync_copy(acc.at[sl], deg_out.at[c, sl])


_NB = 5
_NG = CPT // _NB


@functools.cache
def _edge_agg_kernel():
    @functools.partial(
        pl.kernel,
        out_type=jax.ShapeDtypeStruct((2, R, 64), jnp.float32),
        mesh=_mesh(),
        scratch_types=[
            pltpu.VMEM((CPT, CHUNK), jnp.int32),
            pltpu.VMEM((CPT, CHUNK), jnp.int32),
            pltpu.VMEM((_NB, CHUNK, 64), jnp.float32),
            pltpu.VMEM_SHARED((R, 64), jnp.float32),
        ] + [pltpu.SemaphoreType.DMA] * (2 * _NB),
        compiler_params=_SC_PARAMS,
    )
    def agg(ua, ub, src_hbm, dst_hbm, zeros_hbm, out,
            src_all, dst_all, rows_v, acc, *sems):
        gsem, ssem = sems[:_NB], sems[_NB:]
        c = lax.axis_index("c")
        t = lax.axis_index("s")
        sl = pl.ds(t * ROWS_PER_TILE, ROWS_PER_TILE)
        pltpu.sync_copy(zeros_hbm, acc.at[sl])
        pltpu.sync_copy(src_hbm.at[pl.ds(t * CPT, CPT)], src_all)
        pltpu.sync_copy(dst_hbm.at[pl.ds(t * CPT, CPT)], dst_all)
        plsc.subcore_barrier()

        def run(uref):
            for b in range(_NB):
                pltpu.async_copy(uref.at[src_all.at[b]], rows_v.at[b], gsem[b])

            @pl.loop(0, _NG - 1)
            def _body(g):
                base = g * _NB
                for b in range(_NB):
                    c = base + b
                    pltpu.make_async_copy(uref.at[src_all.at[c]],
                                          rows_v.at[b], gsem[b]).wait()
                    pltpu.async_copy(rows_v.at[b], acc.at[dst_all.at[c]],
                                     ssem[b], add=True)
                    pltpu.make_async_copy(rows_v.at[b],
                                          acc.at[dst_all.at[c]],
                                          ssem[b]).wait()
                    pltpu.async_copy(uref.at[src_all.at[c + _NB]],
                                     rows_v.at[b], gsem[b])

            base = (_NG - 1) * _NB
            sds = []
            for b in range(_NB):
                pltpu.make_async_copy(uref.at[src_all.at[base + b]],
                                      rows_v.at[b], gsem[b]).wait()
                sds.append(pltpu.async_copy(
                    rows_v.at[b], acc.at[dst_all.at[base + b]],
                    ssem[b], add=True))
            for d in sds:
                d.wait()

        @pl.when(c == 0)
        def _():
            run(ua)

        @pl.when(c == 1)
        def _():
            run(ub)

        plsc.subcore_barrier()
        pltpu.sync_copy(acc.at[sl], out.at[c, sl])

    return agg


def _rowmask(i):
    rows = lax.broadcasted_iota(jnp.int32, (B, 1), 0) + i * B
    return (rows < N).astype(jnp.float32)


def _ka_body(x_ref, deg_ref, u1a_ref, u1b_ref, dinv_ref):
    deg = deg_ref[0][:, :1] + deg_ref[1][:, :1] + 1.0
    d = lax.rsqrt(deg)
    u = x_ref[...] * d
    u1a_ref[...] = u[:, :64]
    u1b_ref[...] = u[:, 64:]
    dinv_ref[...] = jnp.broadcast_to(d, (B, 128))


def _stats(m, i):
    msk = _rowmask(i)
    mm = m * msk
    s1 = jnp.sum(mm, axis=0).reshape(1, 1, -1)
    s2 = jnp.sum(mm * m, axis=0).reshape(1, 1, -1)
    return jnp.concatenate([s1, s2], axis=1)


def _bn_consts(p, g_ref, be_ref):
    mu = p[0] / N
    var = p[1] / N - mu * mu
    scale = g_ref[0] * lax.rsqrt(var + 1e-5)
    shift = be_ref[0] - mu * scale
    return scale, shift


def _acc_stats(st_scr, m, i):
    msk = _rowmask(i)
    mm = m * msk
    p = jnp.concatenate([jnp.sum(mm, axis=0).reshape(1, -1),
                         jnp.sum(mm * m, axis=0).reshape(1, -1)], axis=0)
    prev = jnp.where(i == 0, 0.0, st_scr[...])
    st_scr[...] = prev + p


def _make_layer_body(nsq, w2_cols):
    def body(*refs):
        srefs = refs[:nsq]
        urefs_in = refs[nsq:2 * nsq]
        dv, w1_ref, b_ref, g_ref, be_ref, w2_ref = refs[2 * nsq:2 * nsq + 6]
        outs = refs[2 * nsq + 6:2 * nsq + 6 + w2_cols // 64]
        m_scr, st_scr = refs[2 * nsq + 6 + w2_cols // 64:]
        i = pl.program_id(0)

        @pl.when(i < G)
        def _():
            s = jnp.concatenate([r[...] for r in srefs], axis=1)
            u = jnp.concatenate([r[...] for r in urefs_in], axis=1)
            a = (s + u) * dv[:, :1]
            if w1_ref is not None:
                m = jnp.dot(a, w1_ref[...],
                            preferred_element_type=jnp.float32) + b_ref[...]
            else:
                m = a + b_ref[...]
            m_scr[pl.ds(i * B, B), :] = m
            _acc_stats(st_scr, m, i)

        @pl.when(i >= G)
        def _():
            j = i - G
            scale, shift = _bn_consts(st_scr[...], g_ref, be_ref)
            m = m_scr[pl.ds(j * B, B), :]
            h = jnp.maximum(m * scale + shift, 0.0)
            z = jnp.dot(h, w2_ref[...], preferred_element_type=jnp.float32)
            un = z * dv[:, :1] * _rowmask(j)
            for k, ur in enumerate(outs):
                ur[...] = un[:, 64 * k:64 * (k + 1)]

    return body


def _kbc_body(sa, sb, ua, ub, dv, w1_ref, b_ref, g_ref, be_ref, w2_ref,
              o0, o1, o2, o3, m_scr, st_scr):
    _make_layer_body(2, 256)(sa, sb, ua, ub, dv, w1_ref, b_ref, g_ref,
                             be_ref, w2_ref, o0, o1, o2, o3, m_scr, st_scr)


def _kde_body(s0, s1_, s2_, s3_, u0, u1_, u2_, u3_, dv, b_ref, g_ref,
              be_ref, w2_ref, o0, o1, m_scr, st_scr):
    srefs = (s0, s1_, s2_, s3_)
    urefs = (u0, u1_, u2_, u3_)
    i = pl.program_id(0)

    @pl.when(i < G)
    def _():
        s = jnp.concatenate([r[...] for r in srefs], axis=1)
        u = jnp.concatenate([r[...] for r in urefs], axis=1)
        m = (s + u) * dv[:, :1] + b_ref[...]
        m_scr[pl.ds(i * B, B), :] = m
        _acc_stats(st_scr, m, i)

    @pl.when(i >= G)
    def _():
        j = i - G
        scale, shift = _bn_consts(st_scr[...], g_ref, be_ref)
        m = m_scr[pl.ds(j * B, B), :]
        h = jnp.maximum(m * scale + shift, 0.0)
        z = jnp.dot(h, w2_ref[...], preferred_element_type=jnp.float32)
        un = z * dv[:, :1] * _rowmask(j)
        o0[...] = un[:, :64]
        o1[...] = un[:, 64:]


def _kfg_body(sa, sb, ua, ub, dv, b_ref, bt_ref, g_ref, be_ref,
              out_ref, seg_scr, cnt_scr, st_scr):
    i = pl.program_id(0)

    @pl.when(i < G)
    def _():
        s = jnp.concatenate([sa[...], sb[...]], axis=1)
        u = jnp.concatenate([ua[...], ub[...]], axis=1)
        m = (s + u) * dv[:, :1] + b_ref[...]
        _acc_stats(st_scr, m, i)
        bt = bt_ref[:, :1]
        segs, cnts = [], []
        for g in range(16):
            mg = (bt == g).astype(jnp.float32)
            segs.append(jnp.sum(m * mg, axis=0).reshape(1, -1))
            cnts.append(jnp.broadcast_to(
                jnp.sum(mg, axis=0).reshape(1, 1), (1, 128)))
        seg = jnp.concatenate(segs, axis=0)
        cnt = jnp.concatenate(cnts, axis=0)
        seg_scr[...] = jnp.where(i == 0, 0.0, seg_scr[...]) + seg
        cnt_scr[...] = jnp.where(i == 0, 0.0, cnt_scr[...]) + cnt

    @pl.when(i == G)
    def _():
        scale, shift = _bn_consts(st_scr[...], g_ref, be_ref)
        cnt = cnt_scr[...]
        pool = seg_scr[...] / jnp.maximum(cnt, 1.0)
        out_ref[...] = jnp.where(cnt > 0, pool * scale + shift, 0.0)


def _rb(d):
    return pl.BlockSpec((B, d), lambda i: (i, 0))


def _rbw(d):
    return pl.BlockSpec((B, d), lambda i: (jnp.where(i < G, i, i - G), 0))


def _obw(d):
    return pl.BlockSpec((B, d), lambda i: (jnp.where(i < G, 0, i - G), 0))


def _full2(shape):
    return pl.BlockSpec(shape, lambda i: (0, 0))


def _pspec(d):
    return pl.BlockSpec((1, 2, d), lambda i: (i, 0, 0))


def _f32(*shape):
    return jax.ShapeDtypeStruct(shape, jnp.float32)


def kernel(x, edge_index, batch, W1, b1, g1, be1, W2, b2, g2, be2,
           W3, b3, g3, be3):
    f32 = jnp.float32
    src = edge_index[0]
    dst = edge_index[1]

    x_pad = jnp.pad(x, ((0, R - N), (0, 0)))
    batch_pad = jnp.broadcast_to(
        jnp.pad(batch, (0, R - N), constant_values=16)[:, None], (R, 16))

    fill = (N + (jnp.arange(EPAD - E, dtype=jnp.int32) % 16))
    src_p = jnp.concatenate([src, fill]).reshape(NCH, CHUNK)
    dst_p = jnp.concatenate([dst, fill]).reshape(NCH, CHUNK)

    ones8 = jnp.ones((CHUNK, 8), f32)
    z8 = jnp.zeros((ROWS_PER_TILE, 8), f32)
    z64 = jnp.zeros((ROWS_PER_TILE, 64), f32)

    deg2 = _sc_degree_kernel()(dst_p, ones8, z8)

    u1a, u1b, dinv = pl.pallas_call(
        _ka_body,
        grid=(G,),
        in_specs=[_rb(128), pl.BlockSpec((2, B, 8), lambda i: (0, i, 0))],
        out_specs=[_rb(64), _rb(64), _rb(128)],
        out_shape=[_f32(R, 64), _f32(R, 64), _f32(R, 128)],
    )(x_pad, deg2)

    s1 = _edge_agg_kernel()(u1a, u1b, src_p, dst_p, z64)

    wfull = lambda shape: pl.BlockSpec(shape, lambda i: tuple(0 for _ in shape))
    u2q = pl.pallas_call(
        _kbc_body,
        grid=(2 * G,),
        in_specs=[_rbw(64), _rbw(64), _rbw(64), _rbw(64), _rbw(128),
                  wfull((128, 256)), wfull((1, 256)), wfull((1, 256)),
                  wfull((1, 256)), wfull((256, 256))],
        out_specs=[_obw(64)] * 4,
        out_shape=[_f32(R, 64)] * 4,
        scratch_shapes=[pltpu.VMEM((R, 256), jnp.float32),
                        pltpu.VMEM((2, 256), jnp.float32)],
    )(s1[0], s1[1], u1a, u1b, dinv, W1, b1.reshape(1, 256),
      g1.reshape(1, 256), be1.reshape(1, 256), W2)

    s02 = _edge_agg_kernel()(u2q[0], u2q[2], src_p, dst_p, z64)
    s13 = _edge_agg_kernel()(u2q[1], u2q[3], src_p, dst_p, z64)

    u3a, u3b = pl.pallas_call(
        _kde_body,
        grid=(2 * G,),
        in_specs=[_rbw(64)] * 8 + [_rbw(128), wfull((1, 256)),
                  wfull((1, 256)), wfull((1, 256)), wfull((256, 128))],
        out_specs=[_obw(64)] * 2,
        out_shape=[_f32(R, 64)] * 2,
        scratch_shapes=[pltpu.VMEM((R, 256), jnp.float32),
                        pltpu.VMEM((2, 256), jnp.float32)],
    )(s02[0], s13[0], s02[1], s13[1], u2q[0], u2q[1], u2q[2], u2q[3],
      dinv, b2.reshape(1, 256), g2.reshape(1, 256), be2.reshape(1, 256), W3)

    s3 = _edge_agg_kernel()(u3a, u3b, src_p, dst_p, z64)

    kfg_rb = lambda d: pl.BlockSpec((B, d),
                                    lambda i: (jnp.where(i < G, i, 0), 0))
    out = pl.pallas_call(
        _kfg_body,
        grid=(G + 1,),
        in_specs=[kfg_rb(64), kfg_rb(64), kfg_rb(64), kfg_rb(64),
                  kfg_rb(128), wfull((1, 128)), kfg_rb(16),
                  wfull((1, 128)), wfull((1, 128))],
        out_specs=pl.BlockSpec((16, 128), lambda i: (0, 0)),
        out_shape=_f32(16, 128),
        scratch_shapes=[pltpu.VMEM((16, 128), jnp.float32),
                        pltpu.VMEM((16, 128), jnp.float32),
                        pltpu.VMEM((2, 128), jnp.float32)],
    )(s3[0], s3[1], u3a, u3b, dinv, b3.reshape(1, 128), batch_pad,
      g3.reshape(1, 128), be3.reshape(1, 128))

    return out
